# Initial kernel scaffold; baseline (speedup 1.0000x reference)
import jax
import jax.numpy as jnp
from jax.experimental import pallas as pl


def kernel(x, w1, b1, bn1_gamma, bn1_beta, bn1_mean, bn1_var, alpha1, w_dw, b_dw, bn2_gamma, bn2_beta, bn2_mean, bn2_var, alpha2, w3, b3, bn3_gamma, bn3_beta, bn3_mean, bn3_var):
    raise NotImplementedError("write your pallas kernel here")


# speedup vs baseline: 2.1878x; 2.1878x over previous
import functools

import jax
import jax.numpy as jnp
from jax.experimental import pallas as pl
from jax.experimental.pallas import tpu as pltpu

_VMEM_LIMIT = 100 * 1024 * 1024


def _fold_bn(gamma, beta, mean, var, conv_bias, eps=1e-5):
    inv = gamma * jax.lax.rsqrt(var + eps)
    return inv, beta + (conv_bias - mean) * inv


def _fused_body(x_ref, w1_ref, t1_ref, a1_ref, w9_ref, t2_ref, a2_ref,
                w3_ref, t3_ref, o_ref, *, h: int, w: int):
    x2 = x_ref[0]

    y = jnp.dot(x2, w1_ref[...], preferred_element_type=jnp.float32)
    y = y + t1_ref[...]
    y = jnp.maximum(y, 0.0) + a1_ref[...] * jnp.minimum(y, 0.0)

    hidden = y.shape[1]
    hp = jnp.pad(y.reshape(h, w, hidden), ((1, 1), (1, 1), (0, 0)))
    w9 = w9_ref[...]
    acc = t2_ref[...].reshape(1, 1, hidden) * jnp.ones((h, w, hidden), jnp.float32)
    for k in range(9):
        kh, kw = k // 3, k % 3
        acc = acc + hp[kh:kh + h, kw:kw + w, :] * w9[k]
    d = jnp.maximum(acc, 0.0) + a2_ref[...] * jnp.minimum(acc, 0.0)

    o = jnp.dot(d.reshape(h * w, hidden), w3_ref[...],
                preferred_element_type=jnp.float32)
    o_ref[0] = o + t3_ref[...] + x2


def kernel(x, w1, b1, bn1_gamma, bn1_beta, bn1_mean, bn1_var, alpha1,
           w_dw, b_dw, bn2_gamma, bn2_beta, bn2_mean, bn2_var, alpha2,
           w3, b3, bn3_gamma, bn3_beta, bn3_mean, bn3_var):
    n, cin, h, w = x.shape
    hidden = w1.shape[0]
    cout = w3.shape[0]
    hw = h * w

    s1, t1 = _fold_bn(bn1_gamma, bn1_beta, bn1_mean, bn1_var, b1)
    s2, t2 = _fold_bn(bn2_gamma, bn2_beta, bn2_mean, bn2_var, b_dw)
    s3, t3 = _fold_bn(bn3_gamma, bn3_beta, bn3_mean, bn3_var, b3)

    w1f = w1.T * s1[None, :]
    w9 = (jnp.transpose(w_dw[:, 0], (1, 2, 0)).reshape(9, hidden)
          * s2[None, :])
    w3f = w3.T * s3[None, :]
    a1v = jnp.broadcast_to(jnp.asarray(alpha1).reshape(-1), (hidden,)).reshape(1, hidden)
    a2v = jnp.broadcast_to(jnp.asarray(alpha2).reshape(-1), (hidden,)).reshape(1, hidden)

    xt = jnp.transpose(x, (0, 2, 3, 1)).reshape(n, hw, cin)

    out = pl.pallas_call(
        functools.partial(_fused_body, h=h, w=w),
        out_shape=jax.ShapeDtypeStruct((n, hw, cout), x.dtype),
        grid=(n,),
        in_specs=[
            pl.BlockSpec((1, hw, cin), lambda i: (i, 0, 0)),
            pl.BlockSpec((cin, hidden), lambda i: (0, 0)),
            pl.BlockSpec((1, hidden), lambda i: (0, 0)),
            pl.BlockSpec((1, hidden), lambda i: (0, 0)),
            pl.BlockSpec((9, hidden), lambda i: (0, 0)),
            pl.BlockSpec((1, hidden), lambda i: (0, 0)),
            pl.BlockSpec((1, hidden), lambda i: (0, 0)),
            pl.BlockSpec((hidden, cout), lambda i: (0, 0)),
            pl.BlockSpec((1, cout), lambda i: (0, 0)),
        ],
        out_specs=pl.BlockSpec((1, hw, cout), lambda i: (i, 0, 0)),
        compiler_params=pltpu.CompilerParams(
            dimension_semantics=("parallel",),
            vmem_limit_bytes=_VMEM_LIMIT),
    )(xt, w1f, t1.reshape(1, hidden), a1v, w9, t2.reshape(1, hidden), a2v,
      w3f, t3.reshape(1, cout))

    return jnp.transpose(out.reshape(n, h, w, cout), (0, 3, 1, 2))


# speedup vs baseline: 3.3104x; 1.5131x over previous
import functools

import jax
import jax.numpy as jnp
from jax.experimental import pallas as pl
from jax.experimental.pallas import tpu as pltpu

_VMEM_LIMIT = 100 * 1024 * 1024


def _fold_bn(gamma, beta, mean, var, conv_bias, eps=1e-5):
    inv = gamma * jax.lax.rsqrt(var + eps)
    return inv, beta + (conv_bias - mean) * inv


def _fused_body(x_ref, w1_ref, t1_ref, a1_ref, w9_ref, t2_ref, a2_ref,
                w3_ref, t3_ref, o_ref, *, h: int, w: int):
    x2 = x_ref[0]

    y = jnp.dot(x2, w1_ref[...], preferred_element_type=jnp.float32)
    y = y + t1_ref[...]
    y = jnp.where(y >= 0.0, y, a1_ref[...] * y)

    hidden = y.shape[1]
    hp = jnp.pad(y.reshape(h, w, hidden), ((1, 1), (0, 0), (0, 0)))
    w9 = w9_ref[...]
    q = []
    for kw in range(3):
        qk = hp[0:h] * w9[kw]
        qk = qk + hp[1:h + 1] * w9[3 + kw]
        qk = qk + hp[2:h + 2] * w9[6 + kw]
        q.append(qk)
    acc = q[1] + jnp.pad(q[0], ((0, 0), (1, 0), (0, 0)))[:, :w]
    acc = acc + jnp.pad(q[2], ((0, 0), (0, 1), (0, 0)))[:, 1:]
    acc = acc + t2_ref[...]
    d = jnp.where(acc >= 0.0, acc, a2_ref[...] * acc)

    o = jnp.dot(d.reshape(h * w, hidden), w3_ref[...],
                preferred_element_type=jnp.float32)
    o_ref[0] = o + t3_ref[...] + x2


def kernel(x, w1, b1, bn1_gamma, bn1_beta, bn1_mean, bn1_var, alpha1,
           w_dw, b_dw, bn2_gamma, bn2_beta, bn2_mean, bn2_var, alpha2,
           w3, b3, bn3_gamma, bn3_beta, bn3_mean, bn3_var):
    n, cin, h, w = x.shape
    hidden = w1.shape[0]
    cout = w3.shape[0]
    hw = h * w

    s1, t1 = _fold_bn(bn1_gamma, bn1_beta, bn1_mean, bn1_var, b1)
    s2, t2 = _fold_bn(bn2_gamma, bn2_beta, bn2_mean, bn2_var, b_dw)
    s3, t3 = _fold_bn(bn3_gamma, bn3_beta, bn3_mean, bn3_var, b3)

    w1f = w1.T * s1[None, :]
    w9 = (jnp.transpose(w_dw[:, 0], (1, 2, 0)).reshape(9, hidden)
          * s2[None, :])
    w3f = w3.T * s3[None, :]
    a1v = jnp.broadcast_to(jnp.asarray(alpha1).reshape(-1), (hidden,)).reshape(1, hidden)
    a2v = jnp.broadcast_to(jnp.asarray(alpha2).reshape(-1), (hidden,)).reshape(1, hidden)

    xt = jnp.transpose(x, (0, 2, 3, 1)).reshape(n, hw, cin)

    out = pl.pallas_call(
        functools.partial(_fused_body, h=h, w=w),
        out_shape=jax.ShapeDtypeStruct((n, hw, cout), x.dtype),
        grid=(n,),
        in_specs=[
            pl.BlockSpec((1, hw, cin), lambda i: (i, 0, 0)),
            pl.BlockSpec((cin, hidden), lambda i: (0, 0)),
            pl.BlockSpec((1, hidden), lambda i: (0, 0)),
            pl.BlockSpec((1, hidden), lambda i: (0, 0)),
            pl.BlockSpec((9, hidden), lambda i: (0, 0)),
            pl.BlockSpec((1, hidden), lambda i: (0, 0)),
            pl.BlockSpec((1, hidden), lambda i: (0, 0)),
            pl.BlockSpec((hidden, cout), lambda i: (0, 0)),
            pl.BlockSpec((1, cout), lambda i: (0, 0)),
        ],
        out_specs=pl.BlockSpec((1, hw, cout), lambda i: (i, 0, 0)),
        compiler_params=pltpu.CompilerParams(
            dimension_semantics=("parallel",),
            vmem_limit_bytes=_VMEM_LIMIT),
    )(xt, w1f, t1.reshape(1, hidden), a1v, w9, t2.reshape(1, hidden), a2v,
      w3f, t3.reshape(1, cout))

    return jnp.transpose(out.reshape(n, h, w, cout), (0, 3, 1, 2))


# speedup vs baseline: 4.1389x; 1.2503x over previous
import functools

import jax
import jax.numpy as jnp
from jax.experimental import pallas as pl
from jax.experimental.pallas import tpu as pltpu

_VMEM_LIMIT = 100 * 1024 * 1024


def _fold_bn(gamma, beta, mean, var, conv_bias, eps=1e-5):
    inv = gamma * jax.lax.rsqrt(var + eps)
    return inv, beta + (conv_bias - mean) * inv


def _fused_body(x_ref, w1_ref, t1_ref, a1_ref, w9_ref, t2_ref, a2_ref,
                w3_ref, t3_ref, o_ref, *, h: int, w: int):
    x2 = x_ref[0]

    y = jnp.dot(x2, w1_ref[...], preferred_element_type=jnp.float32)
    y = y + t1_ref[...]
    y = jnp.where(y >= 0.0, y, a1_ref[...] * y)

    hidden = y.shape[1]
    yb = y.astype(jnp.bfloat16)
    hp = jnp.pad(yb.reshape(h, w, hidden), ((1, 1), (0, 0), (0, 0)))
    w9 = w9_ref[...]
    q = []
    for kw in range(3):
        qk = hp[0:h] * w9[kw]
        qk = qk + hp[1:h + 1] * w9[3 + kw]
        qk = qk + hp[2:h + 2] * w9[6 + kw]
        q.append(qk)
    acc = q[1] + jnp.pad(q[0], ((0, 0), (1, 0), (0, 0)))[:, :w]
    acc = acc + jnp.pad(q[2], ((0, 0), (0, 1), (0, 0)))[:, 1:]
    acc = acc + t2_ref[...]
    d = jnp.where(acc >= 0, acc, a2_ref[...] * acc)

    o = jnp.dot(d.reshape(h * w, hidden), w3_ref[...],
                preferred_element_type=jnp.float32)
    o_ref[0] = o + t3_ref[...] + x2


def kernel(x, w1, b1, bn1_gamma, bn1_beta, bn1_mean, bn1_var, alpha1,
           w_dw, b_dw, bn2_gamma, bn2_beta, bn2_mean, bn2_var, alpha2,
           w3, b3, bn3_gamma, bn3_beta, bn3_mean, bn3_var):
    n, cin, h, w = x.shape
    hidden = w1.shape[0]
    cout = w3.shape[0]
    hw = h * w

    s1, t1 = _fold_bn(bn1_gamma, bn1_beta, bn1_mean, bn1_var, b1)
    s2, t2 = _fold_bn(bn2_gamma, bn2_beta, bn2_mean, bn2_var, b_dw)
    s3, t3 = _fold_bn(bn3_gamma, bn3_beta, bn3_mean, bn3_var, b3)

    w1f = w1.T * s1[None, :]
    w9 = (jnp.transpose(w_dw[:, 0], (1, 2, 0)).reshape(9, hidden)
          * s2[None, :]).astype(jnp.bfloat16)
    w3f = (w3.T * s3[None, :]).astype(jnp.bfloat16)
    a1v = jnp.broadcast_to(jnp.asarray(alpha1).reshape(-1), (hidden,)).reshape(1, hidden)
    a2v = jnp.broadcast_to(jnp.asarray(alpha2).reshape(-1),
                           (hidden,)).reshape(1, hidden).astype(jnp.bfloat16)
    t2b = t2.reshape(1, hidden).astype(jnp.bfloat16)

    xt = jnp.transpose(x, (0, 2, 3, 1)).reshape(n, hw, cin)

    out = pl.pallas_call(
        functools.partial(_fused_body, h=h, w=w),
        out_shape=jax.ShapeDtypeStruct((n, hw, cout), x.dtype),
        grid=(n,),
        in_specs=[
            pl.BlockSpec((1, hw, cin), lambda i: (i, 0, 0)),
            pl.BlockSpec((cin, hidden), lambda i: (0, 0)),
            pl.BlockSpec((1, hidden), lambda i: (0, 0)),
            pl.BlockSpec((1, hidden), lambda i: (0, 0)),
            pl.BlockSpec((9, hidden), lambda i: (0, 0)),
            pl.BlockSpec((1, hidden), lambda i: (0, 0)),
            pl.BlockSpec((1, hidden), lambda i: (0, 0)),
            pl.BlockSpec((hidden, cout), lambda i: (0, 0)),
            pl.BlockSpec((1, cout), lambda i: (0, 0)),
        ],
        out_specs=pl.BlockSpec((1, hw, cout), lambda i: (i, 0, 0)),
        compiler_params=pltpu.CompilerParams(
            dimension_semantics=("parallel",),
            vmem_limit_bytes=_VMEM_LIMIT),
    )(xt, w1f, t1.reshape(1, hidden), a1v, w9, t2b, a2v,
      w3f, t3.reshape(1, cout))

    return jnp.transpose(out.reshape(n, h, w, cout), (0, 3, 1, 2))


# speedup vs baseline: 4.5783x; 1.1062x over previous
import functools

import jax
import jax.numpy as jnp
from jax.experimental import pallas as pl
from jax.experimental.pallas import tpu as pltpu

_VMEM_LIMIT = 100 * 1024 * 1024


def _fold_bn(gamma, beta, mean, var, conv_bias, eps=1e-5):
    inv = gamma * jax.lax.rsqrt(var + eps)
    return inv, beta + (conv_bias - mean) * inv


def _fused_body(x_ref, w1_ref, t1_ref, a1_ref, w9_ref, t2_ref, a2_ref,
                w3_ref, t3_ref, o_ref, *, h: int, w: int):
    x2 = x_ref[0]

    y = jnp.dot(x2, w1_ref[...], preferred_element_type=jnp.float32)
    yb = y.astype(jnp.bfloat16)
    yb = yb + t1_ref[...]
    yb = jnp.where(yb >= 0, yb, a1_ref[...] * yb)

    hidden = y.shape[1]
    hp = jnp.pad(yb.reshape(h, w, hidden), ((1, 1), (0, 0), (0, 0)))
    w9 = w9_ref[...]
    q = []
    for kw in range(3):
        qk = hp[0:h] * w9[kw]
        qk = qk + hp[1:h + 1] * w9[3 + kw]
        qk = qk + hp[2:h + 2] * w9[6 + kw]
        q.append(qk)
    s = (jnp.pad(q[2], ((0, 0), (0, 2), (0, 0)))
         + jnp.pad(q[0], ((0, 0), (2, 0), (0, 0))))
    acc = q[1] + s[:, 1:w + 1]
    acc = acc + t2_ref[...]
    d = jnp.where(acc >= 0, acc, a2_ref[...] * acc)

    o = jnp.dot(d.reshape(h * w, hidden), w3_ref[...],
                preferred_element_type=jnp.float32)
    o_ref[0] = o + t3_ref[...] + x2


def kernel(x, w1, b1, bn1_gamma, bn1_beta, bn1_mean, bn1_var, alpha1,
           w_dw, b_dw, bn2_gamma, bn2_beta, bn2_mean, bn2_var, alpha2,
           w3, b3, bn3_gamma, bn3_beta, bn3_mean, bn3_var):
    n, cin, h, w = x.shape
    hidden = w1.shape[0]
    cout = w3.shape[0]
    hw = h * w

    s1, t1 = _fold_bn(bn1_gamma, bn1_beta, bn1_mean, bn1_var, b1)
    s2, t2 = _fold_bn(bn2_gamma, bn2_beta, bn2_mean, bn2_var, b_dw)
    s3, t3 = _fold_bn(bn3_gamma, bn3_beta, bn3_mean, bn3_var, b3)

    w1f = w1.T * s1[None, :]
    w9 = (jnp.transpose(w_dw[:, 0], (1, 2, 0)).reshape(9, hidden)
          * s2[None, :]).astype(jnp.bfloat16)
    w3f = (w3.T * s3[None, :]).astype(jnp.bfloat16)
    a1v = jnp.broadcast_to(jnp.asarray(alpha1).reshape(-1),
                           (hidden,)).reshape(1, hidden).astype(jnp.bfloat16)
    a2v = jnp.broadcast_to(jnp.asarray(alpha2).reshape(-1),
                           (hidden,)).reshape(1, hidden).astype(jnp.bfloat16)
    t1b = t1.reshape(1, hidden).astype(jnp.bfloat16)
    t2b = t2.reshape(1, hidden).astype(jnp.bfloat16)

    xt = jnp.transpose(x, (0, 2, 3, 1)).reshape(n, hw, cin)

    out = pl.pallas_call(
        functools.partial(_fused_body, h=h, w=w),
        out_shape=jax.ShapeDtypeStruct((n, hw, cout), x.dtype),
        grid=(n,),
        in_specs=[
            pl.BlockSpec((1, hw, cin), lambda i: (i, 0, 0)),
            pl.BlockSpec((cin, hidden), lambda i: (0, 0)),
            pl.BlockSpec((1, hidden), lambda i: (0, 0)),
            pl.BlockSpec((1, hidden), lambda i: (0, 0)),
            pl.BlockSpec((9, hidden), lambda i: (0, 0)),
            pl.BlockSpec((1, hidden), lambda i: (0, 0)),
            pl.BlockSpec((1, hidden), lambda i: (0, 0)),
            pl.BlockSpec((hidden, cout), lambda i: (0, 0)),
            pl.BlockSpec((1, cout), lambda i: (0, 0)),
        ],
        out_specs=pl.BlockSpec((1, hw, cout), lambda i: (i, 0, 0)),
        compiler_params=pltpu.CompilerParams(
            dimension_semantics=("parallel",),
            vmem_limit_bytes=_VMEM_LIMIT),
    )(xt, w1f, t1b, a1v, w9, t2b, a2v,
      w3f, t3.reshape(1, cout))

    return jnp.transpose(out.reshape(n, h, w, cout), (0, 3, 1, 2))
